# Initial kernel scaffold; baseline (speedup 1.0000x reference)
import jax
import jax.numpy as jnp
from jax.experimental import pallas as pl


def kernel(data, edge_index, W_nenc, W_eenc, W_msg1, W_upd1, W_msg2, W_upd2, W_npred, W_e1, W_e2):
    raise NotImplementedError("write your pallas kernel here")


# speedup vs baseline: 33.6062x; 33.6062x over previous
import functools

import jax
import jax.numpy as jnp
from jax import lax
from jax.experimental import pallas as pl
from jax.experimental.pallas import tpu as pltpu
from jax.experimental.pallas import tpu_sc as plsc

C = 512
H = 64
CW = 528
NC = 2
NS = 16
NW = NC * NS
NSTAT = 10

_BD = 128
_NT = C // _BD


def _sc_stats_kernel(chunk, groups):
    mesh = plsc.VectorSubcoreMesh(core_axis_name="c", subcore_axis_name="s")

    @functools.partial(
        pl.kernel,
        out_type=jax.ShapeDtypeStruct((NSTAT, NW, 16 * CW), jnp.float32),
        mesh=mesh,
        scratch_types=[
            pltpu.VMEM((chunk,), jnp.int32),
            pltpu.VMEM((chunk,), jnp.float32),
            pltpu.VMEM((chunk,), jnp.float32),
            pltpu.VMEM((chunk,), jnp.float32),
            pltpu.VMEM((chunk,), jnp.float32),
        ] + [pltpu.VMEM((16 * CW,), jnp.float32) for _ in range(NSTAT)],
        compiler_params=pltpu.CompilerParams(needs_layout_passes=False),
    )
    def body(x_hbm, y_hbm, z_hbm, v_hbm, ids_hbm, out_hbm,
             idsv, xv, yv, zv, vv, *accs):
        wid = lax.axis_index("s") * NC + lax.axis_index("c")
        base = wid * chunk
        pltpu.sync_copy(ids_hbm.at[pl.ds(base, chunk)], idsv)
        pltpu.sync_copy(x_hbm.at[pl.ds(base, chunk)], xv)
        pltpu.sync_copy(y_hbm.at[pl.ds(base, chunk)], yv)
        pltpu.sync_copy(z_hbm.at[pl.ds(base, chunk)], zv)
        pltpu.sync_copy(v_hbm.at[pl.ds(base, chunk)], vv)

        zero16 = jnp.zeros((16,), jnp.float32)
        ninf16 = jnp.full((16,), -jnp.inf, jnp.float32)
        pinf16 = jnp.full((16,), jnp.inf, jnp.float32)

        def init_body(i, _):
            for k in range(8):
                accs[k][pl.ds(i * 16, 16)] = zero16
            accs[8][pl.ds(i * 16, 16)] = ninf16
            accs[9][pl.ds(i * 16, 16)] = pinf16
            return 0

        lax.fori_loop(0, CW, init_body, 0)

        lanebase = lax.iota(jnp.int32, 16) * CW
        one16 = jnp.ones((16,), jnp.float32)

        def group_body(g, _):
            idv = idsv[pl.ds(g * 16, 16)]
            idxs = [lanebase + idv]
            x16 = xv[pl.ds(g * 16, 16)]
            y16 = yv[pl.ds(g * 16, 16)]
            z16 = zv[pl.ds(g * 16, 16)]
            v16 = vv[pl.ds(g * 16, 16)]
            plsc.addupdate_scatter(accs[0], idxs, one16)
            plsc.addupdate_scatter(accs[1], idxs, x16)
            plsc.addupdate_scatter(accs[2], idxs, y16)
            plsc.addupdate_scatter(accs[3], idxs, z16)
            plsc.addupdate_scatter(accs[4], idxs, x16 * x16)
            plsc.addupdate_scatter(accs[5], idxs, y16 * y16)
            plsc.addupdate_scatter(accs[6], idxs, z16 * z16)
            plsc.addupdate_scatter(accs[7], idxs, v16)
            cmax = plsc.load_gather(accs[8], idxs)
            plsc.store_scatter(accs[8], idxs, jnp.maximum(cmax, v16))
            cmin = plsc.load_gather(accs[9], idxs)
            plsc.store_scatter(accs[9], idxs, jnp.minimum(cmin, v16))
            return 0

        lax.fori_loop(0, groups, group_body, 0)

        for k in range(NSTAT):
            pltpu.sync_copy(accs[k], out_hbm.at[k, wid])

    return body


def _t1_body(p_ref, out_ref):
    cnt = jnp.sum(p_ref[0], axis=0)
    sx = jnp.sum(p_ref[1], axis=0)
    sy = jnp.sum(p_ref[2], axis=0)
    sz = jnp.sum(p_ref[3], axis=0)
    sxx = jnp.sum(p_ref[4], axis=0)
    syy = jnp.sum(p_ref[5], axis=0)
    szz = jnp.sum(p_ref[6], axis=0)
    sv = jnp.sum(p_ref[7], axis=0)
    vmax = jnp.max(p_ref[8], axis=0)
    vmin = jnp.min(p_ref[9], axis=0)

    denom = jnp.maximum(cnt, 1.0)
    inv = 1.0 / denom
    cx = sx * inv
    cy = sy * inv
    cz = sz * inv
    vx = (sxx - 2.0 * cx * sx + cnt * cx * cx) * inv
    vy = (syy - 2.0 * cy * sy + cnt * cy * cy) * inv
    vz = (szz - 2.0 * cz * sz + cnt * cz * cz) * inv
    mval = sv * inv
    lc = jnp.log1p(cnt)
    out_ref[...] = jnp.stack(
        [cx, cy, cz, vx, vy, vz, mval, lc, vmax, vmin], axis=0)


def _t1(partials):
    return pl.pallas_call(
        _t1_body,
        out_shape=jax.ShapeDtypeStruct((NSTAT, CW), jnp.float32),
    )(partials)


def _enc_body(f_ref, w_ref, o_ref):
    o_ref[...] = jax.nn.relu(
        jnp.dot(w_ref[...], f_ref[...], preferred_element_type=jnp.float32))


def _enc(feats_t, wnenc_t):
    return pl.pallas_call(
        _enc_body,
        out_shape=jax.ShapeDtypeStruct((H, C), jnp.float32),
    )(feats_t, wnenc_t)


def _round_body(xt_ref, xdt_ref, ct8_ref, cdt8_ref, cd8_ref, w3t_ref,
                w4t_ref, wmat_ref, wmbt_ref, wmct_ref, wuat_ref, wubt_ref,
                o_ref):
    xt = xt_ref[...]
    xdt = xdt_ref[...]
    ct8 = ct8_ref[...]
    cdt8 = cdt8_ref[...]
    cd8 = cd8_ref[...]

    at = jnp.dot(wmat_ref[...], xt, preferred_element_type=jnp.float32)
    bdt = jnp.dot(wmbt_ref[...], xdt, preferred_element_type=jnp.float32)
    adt = jnp.dot(wmat_ref[...], xdt, preferred_element_type=jnp.float32)

    gx = cd8[:, 0:1] - ct8[0:1, :]
    gy = cd8[:, 1:2] - ct8[1:2, :]
    gz = cd8[:, 2:3] - ct8[2:3, :]
    dist = jnp.sqrt(gx * gx + gy * gy + gz * gz)

    p_full = jnp.dot(w3t_ref[...], ct8, preferred_element_type=jnp.float32)
    pd_cols = jnp.dot(w3t_ref[...], cdt8,
                      preferred_element_type=jnp.float32)

    e = jax.nn.relu(
        pd_cols.reshape(H, _BD, 1) - p_full.reshape(H, 1, C)
        + dist.reshape(1, _BD, C) * w4t_ref[...].reshape(H, 1, 1))
    ec = jnp.dot(wmct_ref[...], e.reshape(H, _BD * C),
                 preferred_element_type=jnp.float32).reshape(H, _BD, C)
    msg = jax.nn.relu(ec + at.reshape(H, 1, C) + bdt.reshape(H, _BD, 1))
    agg = jnp.sum(msg, axis=2) - jax.nn.relu(adt + bdt)

    xn = jax.nn.relu(
        jnp.dot(wuat_ref[...], xdt, preferred_element_type=jnp.float32)
        + jnp.dot(wubt_ref[...], agg, preferred_element_type=jnp.float32))
    o_ref[...] = xn


def _round(x_t, cent_t8, cent8, w3t, w4t, wmat, wmbt, wmct, wuat, wubt):
    full = lambda s: pl.BlockSpec(s, lambda i: (0, 0))
    return pl.pallas_call(
        _round_body,
        grid=(_NT,),
        in_specs=[
            full((H, C)),
            pl.BlockSpec((H, _BD), lambda i: (0, i)),
            full((8, C)),
            pl.BlockSpec((8, _BD), lambda i: (0, i)),
            pl.BlockSpec((_BD, 8), lambda i: (i, 0)),
            full((H, 8)),
            full((H, 1)),
            full((H, H)),
            full((H, H)),
            full((H, H)),
            full((H, H)),
            full((H, H)),
        ],
        out_specs=pl.BlockSpec((H, _BD), lambda i: (0, i)),
        out_shape=jax.ShapeDtypeStruct((H, C), jnp.float32),
    )(x_t, x_t, cent_t8, cent_t8, cent8, w3t, w4t, wmat, wmbt, wmct,
      wuat, wubt)


def _final_body(xt_ref, xst_ref, xs_ref, ct8_ref, cst8_ref, cs8_ref,
                w3t_ref, w4t_ref, we1at_ref, we1bt_ref, we1ct_ref, we2t_ref,
                wnpt_ref, wnp_ref, np_ref, ep_ref):
    xt = xt_ref[...]
    xst = xst_ref[...]
    xs = xs_ref[...]
    ct8 = ct8_ref[...]
    cst8 = cst8_ref[...]
    cs8 = cs8_ref[...]

    b1t = jnp.dot(we1bt_ref[...], xt, preferred_element_type=jnp.float32)
    a1st = jnp.dot(we1at_ref[...], xst, preferred_element_type=jnp.float32)

    npt = jnp.dot(wnpt_ref[...], xt, preferred_element_type=jnp.float32)
    np_ref[...] = npt
    prim1 = npt[1:2, :] > npt[0:1, :]
    prim0 = jnp.logical_not(prim1)

    nps = jnp.dot(xs, wnp_ref[...], preferred_element_type=jnp.float32)
    prim1s = nps[:, 1:2] > nps[:, 0:1]

    p_full = jnp.dot(w3t_ref[...], ct8, preferred_element_type=jnp.float32)
    ps_cols = jnp.dot(w3t_ref[...], cst8,
                      preferred_element_type=jnp.float32)

    gx = cs8[:, 0:1] - ct8[0:1, :]
    gy = cs8[:, 1:2] - ct8[1:2, :]
    gz = cs8[:, 2:3] - ct8[2:3, :]
    dist = jnp.sqrt(gx * gx + gy * gy + gz * gz)

    e = jax.nn.relu(
        p_full.reshape(H, 1, C) - ps_cols.reshape(H, _BD, 1)
        + dist.reshape(1, _BD, C) * w4t_ref[...].reshape(H, 1, 1))
    hc = jnp.dot(we1ct_ref[...], e.reshape(H, _BD * C),
                 preferred_element_type=jnp.float32).reshape(H, _BD, C)
    h = jax.nn.relu(hc + a1st.reshape(H, _BD, 1) + b1t.reshape(H, 1, C))
    ept = jnp.dot(we2t_ref[...], h.reshape(H, _BD * C),
                  preferred_element_type=jnp.float32).reshape(8, _BD, C)

    mask = jnp.logical_and(prim1s.reshape(1, _BD, 1),
                           prim0.reshape(1, 1, C)).astype(jnp.float32)
    ep_ref[...] = (ept * mask).reshape(8, _BD * C)


def _final(x_t, x, cent_t8, cent8, w3t, w4t, we1at, we1bt, we1ct, we2t,
           wnpt, wnpp):
    full = lambda s: pl.BlockSpec(s, lambda j: (0, 0))
    return pl.pallas_call(
        _final_body,
        grid=(_NT,),
        in_specs=[
            full((H, C)),
            pl.BlockSpec((H, _BD), lambda j: (0, j)),
            pl.BlockSpec((_BD, H), lambda j: (j, 0)),
            full((8, C)),
            pl.BlockSpec((8, _BD), lambda j: (0, j)),
            pl.BlockSpec((_BD, 8), lambda j: (j, 0)),
            full((H, 8)),
            full((H, 1)),
            full((H, H)),
            full((H, H)),
            full((H, H)),
            full((8, H)),
            full((8, H)),
            full((H, 8)),
        ],
        out_specs=[
            pl.BlockSpec((8, C), lambda j: (0, 0)),
            pl.BlockSpec((8, _BD * C), lambda j: (0, j)),
        ],
        out_shape=[
            jax.ShapeDtypeStruct((8, C), jnp.float32),
            jax.ShapeDtypeStruct((8, C * C), jnp.float32),
        ],
    )(x_t, x_t, x, cent_t8, cent_t8, cent8, w3t, w4t, we1at, we1bt, we1ct,
      we2t, wnpt, wnpp)


def kernel(data, edge_index, W_nenc, W_eenc, W_msg1, W_upd1, W_msg2, W_upd2,
           W_npred, W_e1, W_e2):
    n = data.shape[0]
    chunk = ((n + NW - 1) // NW + 15) // 16 * 16
    n_pad = chunk * NW
    groups = chunk // 16

    ids = data[:, 5].astype(jnp.int32)
    ids_p = jnp.concatenate(
        [ids, jnp.full((n_pad - n,), C, jnp.int32)])
    zpad = jnp.zeros((n_pad - n,), jnp.float32)
    xc = jnp.concatenate([data[:, 0], zpad])
    yc = jnp.concatenate([data[:, 1], zpad])
    zc = jnp.concatenate([data[:, 2], zpad])
    vc = jnp.concatenate([data[:, 4], zpad])

    partials = _sc_stats_kernel(chunk, groups)(xc, yc, zc, vc, ids_p)
    partials = partials.reshape(NSTAT, NW * 16, CW)

    stats = _t1(partials)

    feats_t = jnp.concatenate(
        [stats[:, :C], jnp.zeros((6, C), jnp.float32)], axis=0)
    cent_t8 = jnp.concatenate(
        [stats[:3, :C], jnp.zeros((5, C), jnp.float32)], axis=0)
    cent8 = cent_t8.T
    wnenc_t = jnp.concatenate(
        [W_nenc.T, jnp.zeros((H, 6), jnp.float32)], axis=1)

    x0_t = _enc(feats_t, wnenc_t)

    w3t = jnp.concatenate(
        [W_eenc[:3].T, jnp.zeros((H, 5), jnp.float32)], axis=1)
    w4t = W_eenc[3:4].T

    def tw(w):
        return w.T

    x1_t = _round(x0_t, cent_t8, cent8, w3t, w4t,
                  tw(W_msg1[:H]), tw(W_msg1[H:2 * H]), tw(W_msg1[2 * H:]),
                  tw(W_upd1[:H]), tw(W_upd1[H:]))
    x2_t = _round(x1_t, cent_t8, cent8, w3t, w4t,
                  tw(W_msg2[:H]), tw(W_msg2[H:2 * H]), tw(W_msg2[2 * H:]),
                  tw(W_upd2[:H]), tw(W_upd2[H:]))

    we2t = jnp.concatenate(
        [W_e2.T, jnp.zeros((6, H), jnp.float32)], axis=0)
    wnpt = jnp.concatenate(
        [W_npred.T, jnp.zeros((6, H), jnp.float32)], axis=0)
    wnpp = jnp.concatenate(
        [W_npred, jnp.zeros((H, 6), jnp.float32)], axis=1)

    npt, ept = _final(x2_t, x2_t.T, cent_t8, cent8, w3t, w4t,
                      tw(W_e1[:H]), tw(W_e1[H:2 * H]), tw(W_e1[2 * H:]),
                      we2t, wnpt, wnpp)

    node_pred = npt[:2].T
    ep2 = ept[:2].T
    edge_pred = (ep2[1:]
                 .reshape(C - 1, C + 1, 2)[:, :C, :]
                 .reshape(C * (C - 1), 2))
    return node_pred, edge_pred
